# baseline (device time: 112860 ns/iter reference)
import jax
import jax.numpy as jnp
from jax import lax
from jax.experimental import pallas as pl
from jax.experimental.pallas import tpu as pltpu

S = 1024
D = 2048
H = 16
DH = 128
DR = 32
DC_SH = 128
N_DEV = 8
SQ = S // N_DEV
NC = 4
CW = D // NC
SCALE = (DH + DR) ** -0.5
F32 = jnp.float32
BF16 = jnp.bfloat16

_DELTAS = [(0, 0, 1), (0, 1, 0), (1, 0, 0),
           (0, 1, 1), (1, 0, 1), (1, 1, 0), (1, 1, 1)]


def _my_pos():
    my_x = lax.axis_index("x")
    my_y = lax.axis_index("y")
    my_z = lax.axis_index("z")
    return my_x, my_y, my_z


def _dot(a, b):
    return jnp.dot(a, b, preferred_element_type=F32)


def _kv_body(x_ref, wdkv_ref, wuk_ref, wuv_ref, wkr_ref, wqr_ref,
             k_ref, v_ref, kr_ref, qr3_ref,
             c_mine, c_peer, wuk_mine, wuk_peer, wuv_mine, wuv_peer,
             send_sems, recv_sems):
    my_x, my_y, my_z = _my_pos()
    peer = (my_x, 1 - my_y, my_z)
    lid = my_x * 4 + my_y * 2 + my_z
    qoff = lid * SQ

    barrier_sem = pltpu.get_barrier_semaphore()
    pl.semaphore_signal(barrier_sem, inc=1, device_id=peer,
                        device_id_type=pl.DeviceIdType.MESH)
    pl.semaphore_wait(barrier_sem, 1)

    wuk_mine[...] = wuk_ref[...].astype(BF16)
    wuv_mine[...] = wuv_ref[...].astype(BF16)
    rdma_wuk = pltpu.make_async_remote_copy(
        src_ref=wuk_mine, dst_ref=wuk_peer,
        send_sem=send_sems.at[1], recv_sem=recv_sems.at[1],
        device_id=peer, device_id_type=pl.DeviceIdType.MESH)
    rdma_wuv = pltpu.make_async_remote_copy(
        src_ref=wuv_mine, dst_ref=wuv_peer,
        send_sem=send_sems.at[2], recv_sem=recv_sems.at[2],
        device_id=peer, device_id_type=pl.DeviceIdType.MESH)
    rdma_wuk.start()
    rdma_wuv.start()

    xb = x_ref[...].astype(BF16)
    c_mine[...] = _dot(xb, wdkv_ref[...].astype(BF16)).astype(BF16)
    rdma_c = pltpu.make_async_remote_copy(
        src_ref=c_mine, dst_ref=c_peer,
        send_sem=send_sems.at[0], recv_sem=recv_sems.at[0],
        device_id=peer, device_id_type=pl.DeviceIdType.MESH)
    rdma_c.start()

    kr_ref[...] = _dot(xb, wkr_ref[...].astype(BF16)).astype(BF16)
    qr = _dot(x_ref[pl.ds(qoff, SQ), :].astype(BF16), wqr_ref[...].astype(BF16))
    for h in range(H):
        qr3_ref[h] = qr[:, h * DR:(h + 1) * DR].astype(BF16)
    k_ref[...] = _dot(c_mine[...], wuk_mine[...]).astype(BF16)
    v_ref[...] = _dot(c_mine[...], wuv_mine[...]).astype(BF16)

    rdma_c.wait()
    rdma_wuk.wait()
    k_ref[...] = (k_ref[...].astype(F32)
                  + _dot(c_peer[...], wuk_peer[...])).astype(BF16)
    rdma_wuv.wait()
    v_ref[...] = (v_ref[...].astype(F32)
                  + _dot(c_peer[...], wuv_peer[...])).astype(BF16)


def _attn_gather_body(xq_ref, wqb_ref, qr3_ref, kr_ref, k_ref, v_ref, wob_ref,
                      out_ref, o_buf, g_ref, sbuf, send_sems, recv_sems):
    my_x, my_y, my_z = _my_pos()
    lid = my_x * 4 + my_y * 2 + my_z

    barrier_sem = pltpu.get_barrier_semaphore()
    for dx, dy, dz in _DELTAS:
        pl.semaphore_signal(
            barrier_sem, inc=1,
            device_id=((my_x + dx) % 2, (my_y + dy) % 2, (my_z + dz) % 2),
            device_id_type=pl.DeviceIdType.MESH)
    pl.semaphore_wait(barrier_sem, len(_DELTAS))

    q_all = _dot(xq_ref[...].astype(BF16), wqb_ref[...]).astype(BF16)
    kr_t = kr_ref[...].T
    for h in range(H):
        hs = slice(h * DH, (h + 1) * DH)
        scores = (_dot(q_all[:, hs], k_ref[:, hs].T)
                  + _dot(qr3_ref[h], kr_t)) * SCALE
        m = jnp.max(scores, axis=-1, keepdims=True)
        p = jnp.exp(scores - m)
        p = p / jnp.sum(p, axis=-1, keepdims=True)
        o_buf[:, hs] = _dot(p.astype(BF16), v_ref[:, hs]).astype(BF16)

    rdmas = []
    for c in range(NC):
        cs = pl.ds(c * CW, CW)
        sbuf[c] = _dot(o_buf[...], wob_ref[:, cs]).astype(BF16)
        for j, (dx, dy, dz) in enumerate(_DELTAS):
            peer = ((my_x + dx) % 2, (my_y + dy) % 2, (my_z + dz) % 2)
            r = pltpu.make_async_remote_copy(
                src_ref=sbuf.at[c], dst_ref=g_ref.at[lid, :, cs],
                send_sem=send_sems.at[j * NC + c],
                recv_sem=recv_sems.at[j * NC + c],
                device_id=peer, device_id_type=pl.DeviceIdType.MESH)
            r.start()
            rdmas.append(r)
        g_ref[lid, :, cs] = sbuf[c]

    for r in rdmas:
        r.wait_recv()
    out_ref[...] = g_ref[...].astype(F32)
    for r in rdmas:
        r.wait_send()


def kernel(x, Wdkv, Wuk, Wuv, Wq, Wqr, Wkr, Wo):
    x2 = x.reshape(S, D)

    k, v, kr, qr3 = pl.pallas_call(
        _kv_body,
        out_shape=[
            jax.ShapeDtypeStruct((S, D), BF16),
            jax.ShapeDtypeStruct((S, D), BF16),
            jax.ShapeDtypeStruct((S, DR), BF16),
            jax.ShapeDtypeStruct((H, SQ, DR), BF16),
        ],
        in_specs=[pl.BlockSpec(memory_space=pltpu.VMEM)] * 6,
        out_specs=[pl.BlockSpec(memory_space=pltpu.VMEM)] * 4,
        scratch_shapes=[
            pltpu.VMEM((S, DC_SH), BF16),
            pltpu.VMEM((S, DC_SH), BF16),
            pltpu.VMEM((DC_SH, D), BF16),
            pltpu.VMEM((DC_SH, D), BF16),
            pltpu.VMEM((DC_SH, D), BF16),
            pltpu.VMEM((DC_SH, D), BF16),
            pltpu.SemaphoreType.DMA((3,)),
            pltpu.SemaphoreType.DMA((3,)),
        ],
        compiler_params=pltpu.CompilerParams(
            collective_id=0, vmem_limit_bytes=60 * 2**20),
    )(x2, Wdkv, Wuk, Wuv, Wkr, Wqr)

    wqb = Wq.astype(BF16)
    wob = Wo.astype(BF16)

    my_x = lax.axis_index("x")
    my_y = lax.axis_index("y")
    my_z = lax.axis_index("z")
    lid = my_x * 4 + my_y * 2 + my_z
    xq = lax.dynamic_slice(x2, (lid * SQ, 0), (SQ, D))

    out = pl.pallas_call(
        _attn_gather_body,
        out_shape=jax.ShapeDtypeStruct((N_DEV, SQ, D), F32),
        in_specs=[pl.BlockSpec(memory_space=pltpu.VMEM)] * 7,
        out_specs=pl.BlockSpec(memory_space=pltpu.VMEM),
        scratch_shapes=[
            pltpu.VMEM((SQ, D), BF16),
            pltpu.VMEM((N_DEV, SQ, D), BF16),
            pltpu.VMEM((NC, SQ, CW), BF16),
            pltpu.SemaphoreType.DMA((len(_DELTAS) * NC,)),
            pltpu.SemaphoreType.DMA((len(_DELTAS) * NC,)),
        ],
        compiler_params=pltpu.CompilerParams(
            collective_id=1, vmem_limit_bytes=60 * 2**20),
    )(xq, wqb, qr3, kr, k, v, wob)

    return out.reshape(1, S, D)


# device time: 103175 ns/iter; 1.0939x vs baseline; 1.0939x over previous
import jax
import jax.numpy as jnp
from jax import lax
from jax.experimental import pallas as pl
from jax.experimental.pallas import tpu as pltpu

S = 1024
D = 2048
H = 16
DH = 128
DR = 32
DC_SH = 128
N_DEV = 8
SQ = S // N_DEV
HB = 4
SCALE = (DH + DR) ** -0.5
F32 = jnp.float32
BF16 = jnp.bfloat16

_DELTAS = [(0, 0, 1), (0, 1, 0), (1, 0, 0),
           (0, 1, 1), (1, 0, 1), (1, 1, 0), (1, 1, 1)]


def _my_pos():
    my_x = lax.axis_index("x")
    my_y = lax.axis_index("y")
    my_z = lax.axis_index("z")
    return my_x, my_y, my_z


def _dot(a, b):
    return jnp.dot(a, b, preferred_element_type=F32)


def _kv_body(x_ref, wdkv_ref, wuk_ref, wuv_ref, wkr_ref, wqr_ref,
             k_ref, v_ref, kr_ref, qr3_ref,
             c_mine, c_peer, wuk_mine, wuk_peer, wuv_mine, wuv_peer,
             send_sems, recv_sems):
    my_x, my_y, my_z = _my_pos()
    peer = (my_x, 1 - my_y, my_z)
    lid = my_x * 4 + my_y * 2 + my_z
    qoff = lid * SQ

    barrier_sem = pltpu.get_barrier_semaphore()
    pl.semaphore_signal(barrier_sem, inc=1, device_id=peer,
                        device_id_type=pl.DeviceIdType.MESH)
    pl.semaphore_wait(barrier_sem, 1)

    wuk_mine[...] = wuk_ref[...].astype(BF16)
    wuv_mine[...] = wuv_ref[...].astype(BF16)
    rdma_wuk = pltpu.make_async_remote_copy(
        src_ref=wuk_mine, dst_ref=wuk_peer,
        send_sem=send_sems.at[1], recv_sem=recv_sems.at[1],
        device_id=peer, device_id_type=pl.DeviceIdType.MESH)
    rdma_wuv = pltpu.make_async_remote_copy(
        src_ref=wuv_mine, dst_ref=wuv_peer,
        send_sem=send_sems.at[2], recv_sem=recv_sems.at[2],
        device_id=peer, device_id_type=pl.DeviceIdType.MESH)
    rdma_wuk.start()
    rdma_wuv.start()

    xb = x_ref[...].astype(BF16)
    c_mine[...] = _dot(xb, wdkv_ref[...].astype(BF16)).astype(BF16)
    rdma_c = pltpu.make_async_remote_copy(
        src_ref=c_mine, dst_ref=c_peer,
        send_sem=send_sems.at[0], recv_sem=recv_sems.at[0],
        device_id=peer, device_id_type=pl.DeviceIdType.MESH)
    rdma_c.start()

    kr_ref[...] = _dot(xb, wkr_ref[...].astype(BF16)).astype(BF16)
    qr = _dot(x_ref[pl.ds(qoff, SQ), :].astype(BF16), wqr_ref[...].astype(BF16))
    for h in range(H):
        qr3_ref[h] = qr[:, h * DR:(h + 1) * DR].astype(BF16)
    k_ref[...] = _dot(c_mine[...], wuk_mine[...]).astype(BF16)
    v_ref[...] = _dot(c_mine[...], wuv_mine[...]).astype(BF16)

    rdma_c.wait()
    rdma_wuk.wait()
    k_ref[...] = (k_ref[...].astype(F32)
                  + _dot(c_peer[...], wuk_peer[...])).astype(BF16)
    rdma_wuv.wait()
    v_ref[...] = (v_ref[...].astype(F32)
                  + _dot(c_peer[...], wuv_peer[...])).astype(BF16)


def _attn_body(x_ref, wq_ref, qr3_ref, kr_ref, k_ref, v_ref, wo_ref, out_ref):
    g = pl.program_id(0)
    my_x, my_y, my_z = _my_pos()
    qoff = (my_x * 4 + my_y * 2 + my_z) * SQ

    q4 = _dot(x_ref[pl.ds(qoff, SQ), :].astype(BF16),
              wq_ref[...].astype(BF16)).astype(BF16)
    k4 = k_ref[...]
    v4 = v_ref[...]
    kr_t = kr_ref[...].T
    os = []
    for i in range(HB):
        qi = q4[:, i * DH:(i + 1) * DH]
        ki = k4[:, i * DH:(i + 1) * DH]
        scores = (_dot(qi, ki.T) + _dot(qr3_ref[i], kr_t)) * SCALE
        m = jnp.max(scores, axis=-1, keepdims=True)
        p = jnp.exp(scores - m)
        p = p / jnp.sum(p, axis=-1, keepdims=True)
        os.append(_dot(p.astype(BF16),
                       v4[:, i * DH:(i + 1) * DH]).astype(BF16))
    contrib = _dot(jnp.concatenate(os, axis=1), wo_ref[...].astype(BF16))

    @pl.when(g == 0)
    def _():
        out_ref[...] = jnp.zeros_like(out_ref)

    out_ref[...] += contrib


def _gather_body(oq_ref, out_ref, g_ref, sbuf, send_sems, recv_sems):
    my_x, my_y, my_z = _my_pos()
    lid = my_x * 4 + my_y * 2 + my_z
    sbuf[...] = oq_ref[...].astype(BF16)

    barrier_sem = pltpu.get_barrier_semaphore()
    for dx, dy, dz in _DELTAS:
        pl.semaphore_signal(
            barrier_sem, inc=1,
            device_id=((my_x + dx) % 2, (my_y + dy) % 2, (my_z + dz) % 2),
            device_id_type=pl.DeviceIdType.MESH)
    pl.semaphore_wait(barrier_sem, len(_DELTAS))

    rdmas = []
    for j, (dx, dy, dz) in enumerate(_DELTAS):
        peer = ((my_x + dx) % 2, (my_y + dy) % 2, (my_z + dz) % 2)
        r = pltpu.make_async_remote_copy(
            src_ref=sbuf, dst_ref=g_ref.at[lid],
            send_sem=send_sems.at[j], recv_sem=recv_sems.at[j],
            device_id=peer, device_id_type=pl.DeviceIdType.MESH)
        r.start()
        rdmas.append(r)

    g_ref[lid] = sbuf[...]
    for r in rdmas:
        r.wait_recv()
    for j in range(N_DEV):
        out_ref[j * SQ:(j + 1) * SQ, :] = g_ref[j].astype(F32)
    for r in rdmas:
        r.wait_send()


def kernel(x, Wdkv, Wuk, Wuv, Wq, Wqr, Wkr, Wo):
    x2 = x.reshape(S, D)

    k, v, kr, qr3 = pl.pallas_call(
        _kv_body,
        out_shape=[
            jax.ShapeDtypeStruct((S, D), BF16),
            jax.ShapeDtypeStruct((S, D), BF16),
            jax.ShapeDtypeStruct((S, DR), BF16),
            jax.ShapeDtypeStruct((H, SQ, DR), BF16),
        ],
        in_specs=[pl.BlockSpec(memory_space=pltpu.VMEM)] * 6,
        out_specs=[pl.BlockSpec(memory_space=pltpu.VMEM)] * 4,
        scratch_shapes=[
            pltpu.VMEM((S, DC_SH), BF16),
            pltpu.VMEM((S, DC_SH), BF16),
            pltpu.VMEM((DC_SH, D), BF16),
            pltpu.VMEM((DC_SH, D), BF16),
            pltpu.VMEM((DC_SH, D), BF16),
            pltpu.VMEM((DC_SH, D), BF16),
            pltpu.SemaphoreType.DMA((3,)),
            pltpu.SemaphoreType.DMA((3,)),
        ],
        compiler_params=pltpu.CompilerParams(
            collective_id=0, vmem_limit_bytes=60 * 2**20),
    )(x2, Wdkv, Wuk, Wuv, Wkr, Wqr)

    oq = pl.pallas_call(
        _attn_body,
        grid=(H // HB,),
        out_shape=jax.ShapeDtypeStruct((SQ, D), F32),
        in_specs=[
            pl.BlockSpec((S, D), lambda g: (0, 0)),
            pl.BlockSpec((D, HB * DH), lambda g: (0, g)),
            pl.BlockSpec((HB, SQ, DR), lambda g: (g, 0, 0)),
            pl.BlockSpec((S, DR), lambda g: (0, 0)),
            pl.BlockSpec((S, HB * DH), lambda g: (0, g)),
            pl.BlockSpec((S, HB * DH), lambda g: (0, g)),
            pl.BlockSpec((HB * DH, D), lambda g: (g, 0)),
        ],
        out_specs=pl.BlockSpec((SQ, D), lambda g: (0, 0)),
        compiler_params=pltpu.CompilerParams(
            dimension_semantics=("arbitrary",),
        ),
    )(x2, Wq, qr3, kr, k, v, Wo)

    out = pl.pallas_call(
        _gather_body,
        out_shape=jax.ShapeDtypeStruct((S, D), F32),
        in_specs=[pl.BlockSpec(memory_space=pltpu.VMEM)],
        out_specs=pl.BlockSpec(memory_space=pltpu.VMEM),
        scratch_shapes=[
            pltpu.VMEM((N_DEV, SQ, D), BF16),
            pltpu.VMEM((SQ, D), BF16),
            pltpu.SemaphoreType.DMA((len(_DELTAS),)),
            pltpu.SemaphoreType.DMA((len(_DELTAS),)),
        ],
        compiler_params=pltpu.CompilerParams(
            collective_id=1, vmem_limit_bytes=60 * 2**20),
    )(oq)

    return out.reshape(1, S, D)
